# Initial kernel scaffold; baseline (speedup 1.0000x reference)
import jax
import jax.numpy as jnp
from jax.experimental import pallas as pl


def kernel(pred_tensor, target_tensor):
    raise NotImplementedError("write your pallas kernel here")


# speedup vs baseline: 10.6357x; 10.6357x over previous
import jax
import jax.numpy as jnp
from jax.experimental import pallas as pl
from jax.experimental.pallas import tpu as pltpu

_S = 7.0
_N = 90
_LC = 5.0
_LN = 0.5
_BATCH = 4096
_M = _BATCH * 49
_BB = 2048
_STEPS = _M // _BB


def _body(p_ref, t_ref, o_ref):
    i = pl.program_id(0)
    p = p_ref[...]
    t = t_ref[...]

    dcls = p[:, 10:] - t[:, 10:]
    obj_col = (t[:, 4:5] > 0.0).astype(jnp.float32)
    class_sum = jnp.sum(jnp.sum(dcls * dcls, axis=1, keepdims=True) * obj_col)

    P = jnp.transpose(p[:, :16])
    T = jnp.transpose(t[:, :16])

    pwh0 = P[2:4]
    pwh1 = P[7:9]
    p1a = P[0:2] / _S - 0.5 * pwh0
    p2a = p1a / _S + 0.5 * pwh0
    p1b = P[5:7] / _S - 0.5 * pwh1
    p2b = p1b / _S + 0.5 * pwh1
    twh = T[2:4]
    t1 = T[0:2] / _S - 0.5 * twh
    t2 = t1 / _S + 0.5 * twh

    def iou(b1, b2):
        tl = jnp.maximum(b1, t1)
        br = jnp.minimum(b2, t2)
        wh = jnp.maximum(br - tl, 0.0)
        inter = wh[0:1] * wh[1:2]
        ap = (b2[0:1] - b1[0:1]) * (b2[1:2] - b1[1:2])
        at = (t2[0:1] - t1[0:1]) * (t2[1:2] - t1[1:2])
        return inter / (ap + at - inter)

    iou0 = iou(p1a, p2a)
    iou1 = iou(p1b, p2b)
    sel = iou1 > iou0
    max_iou = jnp.where(sel, iou1, iou0)
    rp1 = jnp.where(sel, p1b, p1a)
    rp2 = jnp.where(sel, p2b, p2a)
    rpc = jnp.where(sel, P[9:10], P[4:5])
    rt1 = jnp.where(sel, T[5:7], t1)
    rt2 = jnp.where(sel, T[7:9], t2)

    objT = (T[4:5] > 0.0).astype(jnp.float32)
    d1 = rp1 - rt1
    d2 = rp2 - rt2
    xywh_se = jnp.sum(d1 * d1, axis=0, keepdims=True) + jnp.sum(d2 * d2, axis=0, keepdims=True)
    cc = rpc - max_iou
    box_sum = jnp.sum(objT * (_LC * xywh_se + cc * cc))

    noobjT = (T[4:5] == 0.0).astype(jnp.float32)
    c0 = P[4:5] - T[4:5]
    c1 = P[9:10] - T[9:10]
    noobj_sum = jnp.sum(noobjT * (c0 * c0 + c1 * c1))

    partial = class_sum + box_sum + _LN * noobj_sum

    @pl.when(i == 0)
    def _():
        o_ref[0, 0] = 0.0

    o_ref[0, 0] += partial

    @pl.when(i == _STEPS - 1)
    def _():
        o_ref[0, 0] = o_ref[0, 0] * (1.0 / _BATCH)


def kernel(pred_tensor, target_tensor):
    p2 = pred_tensor.reshape(_M, _N)
    t2 = target_tensor.reshape(_M, _N)
    out = pl.pallas_call(
        _body,
        grid=(_STEPS,),
        in_specs=[
            pl.BlockSpec((_BB, _N), lambda i: (i, 0)),
            pl.BlockSpec((_BB, _N), lambda i: (i, 0)),
        ],
        out_specs=pl.BlockSpec(memory_space=pltpu.SMEM),
        out_shape=jax.ShapeDtypeStruct((1, 1), jnp.float32),
        compiler_params=pltpu.CompilerParams(
            dimension_semantics=("arbitrary",),
        ),
    )(p2, t2)
    return out[0, 0]


# speedup vs baseline: 11.4339x; 1.0751x over previous
import functools

import jax
import jax.numpy as jnp
from jax import lax
from jax.experimental import pallas as pl
from jax.experimental.pallas import tpu as pltpu
from jax.experimental.pallas import tpu_sc as plsc

_S = 7.0
_N = 90
_LC = 5.0
_LN = 0.5
_BATCH = 4096
_M = _BATCH * 49
_NW = 32
_CELLS_W = _M // _NW
_CHUNK = 224
_NCH = _CELLS_W // _CHUNK
_CW = _CHUNK * _N
_GRP = _CHUNK // 16

_mesh = plsc.VectorSubcoreMesh(core_axis_name="c", subcore_axis_name="s")


@functools.partial(
    pl.kernel,
    out_type=jax.ShapeDtypeStruct((_NW, 16), jnp.float32),
    mesh=_mesh,
    scratch_types=[
        pltpu.VMEM((_CW,), jnp.float32),
        pltpu.VMEM((_CW,), jnp.float32),
        pltpu.VMEM((16,), jnp.float32),
    ],
    compiler_params=pltpu.CompilerParams(needs_layout_passes=False),
)
def _sc_loss(p_hbm, t_hbm, out_hbm, pv, tv, acc_v):
    wid = lax.axis_index("s") * 2 + lax.axis_index("c")
    iota = lax.broadcasted_iota(jnp.int32, (16,), 0)
    idx90 = iota * _N
    zero = jnp.zeros((16,), jnp.float32)

    def chunk_body(k, acc):
        start = (wid * _CELLS_W + k * _CHUNK) * _N
        pltpu.sync_copy(p_hbm.at[pl.ds(start, _CW)], pv)
        pltpu.sync_copy(t_hbm.at[pl.ds(start, _CW)], tv)

        def group_body(g, a):
            gidx = g * (16 * _N) + idx90

            def gp(c):
                return plsc.load_gather(pv, [gidx + c])

            def gt(c):
                return plsc.load_gather(tv, [gidx + c])

            t4 = gt(4)
            obj = (t4 > 0.0).astype(jnp.float32)

            cls = zero
            for c in range(10, _N):
                d = gp(c) - gt(c)
                cls = cls + d * d

            p0, p1, p2, p3, p4 = gp(0), gp(1), gp(2), gp(3), gp(4)
            p5, p6, p7, p8, p9 = gp(5), gp(6), gp(7), gp(8), gp(9)
            t0, t1, t2, t3 = gt(0), gt(1), gt(2), gt(3)
            t5, t6, t7, t8, t9 = gt(5), gt(6), gt(7), gt(8), gt(9)

            ax1 = p0 / _S - 0.5 * p2
            ay1 = p1 / _S - 0.5 * p3
            ax2 = ax1 / _S + 0.5 * p2
            ay2 = ay1 / _S + 0.5 * p3
            bx1 = p5 / _S - 0.5 * p7
            by1 = p6 / _S - 0.5 * p8
            bx2 = bx1 / _S + 0.5 * p7
            by2 = by1 / _S + 0.5 * p8
            tx1 = t0 / _S - 0.5 * t2
            ty1 = t1 / _S - 0.5 * t3
            tx2 = tx1 / _S + 0.5 * t2
            ty2 = ty1 / _S + 0.5 * t3
            at = (tx2 - tx1) * (ty2 - ty1)

            def iou(x1, y1, x2, y2):
                wx = jnp.maximum(jnp.minimum(x2, tx2) - jnp.maximum(x1, tx1), 0.0)
                wy = jnp.maximum(jnp.minimum(y2, ty2) - jnp.maximum(y1, ty1), 0.0)
                inter = wx * wy
                ap = (x2 - x1) * (y2 - y1)
                return inter / (ap + at - inter)

            iou0 = iou(ax1, ay1, ax2, ay2)
            iou1 = iou(bx1, by1, bx2, by2)
            sel = iou1 > iou0
            max_iou = jnp.where(sel, iou1, iou0)

            dx1 = jnp.where(sel, bx1, ax1) - jnp.where(sel, t5, tx1)
            dy1 = jnp.where(sel, by1, ay1) - jnp.where(sel, t6, ty1)
            dx2 = jnp.where(sel, bx2, ax2) - jnp.where(sel, t7, tx2)
            dy2 = jnp.where(sel, by2, ay2) - jnp.where(sel, t8, ty2)
            rpc = jnp.where(sel, p9, p4)

            xywh = dx1 * dx1 + dy1 * dy1 + dx2 * dx2 + dy2 * dy2
            cc = rpc - max_iou
            d4 = p4 - t4
            d9 = p9 - t9
            conf = d4 * d4 + d9 * d9

            contrib = obj * (cls + _LC * xywh + cc * cc) + (_LN * (1.0 - obj)) * conf
            return a + contrib

        return lax.fori_loop(0, _GRP, group_body, acc)

    acc = lax.fori_loop(0, _NCH, chunk_body, zero)
    acc_v[...] = acc
    pltpu.sync_copy(acc_v, out_hbm.at[wid])


def kernel(pred_tensor, target_tensor):
    pf = pred_tensor.reshape(-1)
    tf = target_tensor.reshape(-1)
    parts = _sc_loss(pf, tf)
    return jnp.sum(parts) * (1.0 / _BATCH)
